# Initial kernel scaffold; baseline (speedup 1.0000x reference)
import jax
import jax.numpy as jnp
from jax.experimental import pallas as pl


def kernel(x, edge_index, batch, W_rel, b_rel, W_root, W_lin, b_lin):
    raise NotImplementedError("write your pallas kernel here")


# speedup vs baseline: 12.0018x; 12.0018x over previous
import functools

import jax
import jax.numpy as jnp
from jax import lax
from jax.experimental import pallas as pl
from jax.experimental.pallas import tpu as pltpu
from jax.experimental.pallas import tpu_sc as plsc

G = 64
NC = 2
NS = 16
CH = 128


def _proj_body(x_ref, wrel_ref, wroot_ref, yrel_ref, yroot_ref):
    x = x_ref[...]
    yrel_ref[...] = jnp.dot(x, wrel_ref[...], preferred_element_type=jnp.float32)
    yroot_ref[...] = jnp.dot(x, wroot_ref[...], preferred_element_type=jnp.float32)


def _make_edge_agg(n_acc, chunks):
    rows_per_tile_acc = n_acc // NS
    mesh = plsc.VectorSubcoreMesh(core_axis_name="c", subcore_axis_name="s")

    @functools.partial(
        pl.kernel,
        out_type=jax.ShapeDtypeStruct((NC * n_acc, 16), jnp.float32),
        mesh=mesh,
        compiler_params=pltpu.CompilerParams(use_tc_tiling_on_sc=False),
        scratch_types=[
            pltpu.VMEM((chunks, CH), jnp.int32),
            pltpu.VMEM((chunks, CH), jnp.int32),
            pltpu.VMEM((CH, 16), jnp.float32),
            pltpu.VMEM((rows_per_tile_acc, 16), jnp.float32),
            pltpu.VMEM_SHARED((n_acc, 16), jnp.float32),
        ],
    )
    def edge_agg(yrel_hbm, src_hbm, dst_hbm, out_hbm,
                 src_v, dst_v, buf_v, stage_v, acc_sh):
        c = lax.axis_index("c")
        s = lax.axis_index("s")
        w = s * NC + c

        def zbody(i, carry):
            stage_v[i] = jnp.zeros((16,), jnp.float32)
            return carry
        lax.fori_loop(0, rows_per_tile_acc, zbody, 0)
        pltpu.sync_copy(stage_v, acc_sh.at[pl.ds(s * rows_per_tile_acc,
                                                 rows_per_tile_acc)])
        plsc.subcore_barrier()

        pltpu.sync_copy(src_hbm.at[w], src_v)
        pltpu.sync_copy(dst_hbm.at[w], dst_v)

        def body(j, carry):
            pltpu.sync_copy(yrel_hbm.at[src_v.at[j]], buf_v)
            pltpu.sync_copy(buf_v, acc_sh.at[dst_v.at[j]], add=True)
            return carry
        lax.fori_loop(0, chunks, body, 0)
        plsc.subcore_barrier()

        base = s * rows_per_tile_acc
        pltpu.sync_copy(acc_sh.at[pl.ds(base, rows_per_tile_acc)], stage_v)
        pltpu.sync_copy(stage_v,
                        out_hbm.at[pl.ds(c * n_acc + base, rows_per_tile_acc)])

    return edge_agg


def _make_finish(n, n_acc):
    def finish_body(parts_ref, yroot_ref, batch_ref, brel_ref, wlin_ref,
                    blin_ref, out_ref):
        parts = parts_ref[...]
        h = parts[:n] + parts[n_acc:n_acc + n] + yroot_ref[...] + brel_ref[...]
        h = jnp.maximum(h, 0.0)
        gids = lax.broadcasted_iota(jnp.int32, (1, G), 1)
        onehot = (batch_ref[...] == gids).astype(jnp.float32)
        pooled = lax.dot_general(onehot, h, (((0,), (0,)), ((), ())))
        ones = jnp.ones((n, 1), jnp.float32)
        cnt = lax.dot_general(onehot, ones, (((0,), (0,)), ((), ())))
        pooled = jnp.maximum(pooled / jnp.maximum(cnt, 1.0), 0.0)
        out_ref[...] = (jnp.dot(pooled, wlin_ref[...],
                                preferred_element_type=jnp.float32)
                        + blin_ref[...])
    return finish_body


def kernel(x, edge_index, batch, W_rel, b_rel, W_root, W_lin, b_lin):
    n, d = x.shape
    e = edge_index.shape[1]
    h_dim = W_rel.shape[0]
    c_dim = W_lin.shape[0]

    yrel, yroot = pl.pallas_call(
        _proj_body,
        out_shape=[jax.ShapeDtypeStruct((n, h_dim), jnp.float32),
                   jax.ShapeDtypeStruct((n, h_dim), jnp.float32)],
    )(x, W_rel.T, W_root.T)

    ept = -(-e // (NC * NS * CH)) * CH
    chunks = ept // CH
    e_pad = NC * NS * ept
    n_acc = -(-(n + 1) // (NS * 8)) * (NS * 8)
    src = edge_index[0]
    dst = edge_index[1]
    if e_pad != e:
        src = jnp.concatenate([src, jnp.zeros((e_pad - e,), jnp.int32)])
        dst = jnp.concatenate([dst, jnp.full((e_pad - e,), n, jnp.int32)])
    src3 = src.reshape(NC * NS, chunks, CH)
    dst3 = dst.reshape(NC * NS, chunks, CH)
    parts = _make_edge_agg(n_acc, chunks)(yrel, src3, dst3)

    out = pl.pallas_call(
        _make_finish(n, n_acc),
        out_shape=jax.ShapeDtypeStruct((G, c_dim), jnp.float32),
    )(parts, yroot, batch.reshape(n, 1), b_rel.reshape(1, h_dim),
      W_lin.T, b_lin.reshape(1, c_dim))
    return out
